# baseline (speedup 1.0000x reference)
import jax
import jax.numpy as jnp
from jax.experimental import pallas as pl
from jax.experimental.pallas import tpu as pltpu

_N = 10000
_B = 4
_TOPK = 5000
_ROWS1 = 400
_ROWS2 = 320


def _scores_body(g_ref, k_ref, s_ref):
    s_ref[0, :, :] = jax.lax.dot_general(
        g_ref[...], k_ref[...], (((1,), (0,)), ((), ())),
        preferred_element_type=jnp.float32)


def _mask_xl_body(s_ref, x_ref, mask_ref, xl_ref):
    s = s_ref[...]
    x = x_ref[...]
    key = jax.lax.bitcast_convert_type(s, jnp.int32)
    key = jnp.where(key < 0, key ^ jnp.int32(0x7FFFFFFF), key)
    idx = jax.lax.broadcasted_iota(jnp.int32, (1, _N), 1)
    kk = jnp.int32(_TOPK)

    cnt_nonneg = jnp.sum((key >= 0).astype(jnp.int32))
    pos = cnt_nonneg >= kk
    lo0 = jnp.where(pos, jnp.int32(0), jnp.int32(-2147483648))
    hi0 = jnp.where(pos, jnp.int32(2147483647), jnp.int32(-1))

    def body(_, lh):
        lo, hi = lh
        d = hi - lo
        mid = lo + (d >> 1) + (d & 1)
        c = jnp.sum((key >= mid).astype(jnp.int32))
        big = c >= kk
        return (jnp.where(big, mid, lo), jnp.where(big, hi, mid - 1))

    lo, _hi = jax.lax.fori_loop(0, 31, body, (lo0, hi0))
    theta = lo
    gt = key > theta
    eq = key == theta
    need = kk - jnp.sum(gt.astype(jnp.int32))

    def body2(_, lh):
        lo2, hi2 = lh
        mid = lo2 + ((hi2 - lo2) >> 1)
        c = jnp.sum((eq & (idx <= mid)).astype(jnp.int32))
        ok = c >= need
        return (jnp.where(ok, lo2, mid + 1), jnp.where(ok, mid, hi2))

    m, _m2 = jax.lax.fori_loop(0, 14, body2,
                               (jnp.int32(0), jnp.int32(_N - 1)))
    mask = gt | (eq & (idx <= m))
    mask_ref[...] = mask.astype(jnp.float32)
    xl_ref[...] = jnp.sum(x * s, axis=1, keepdims=True)


def _mask_mul_body(g_ref, m_ref, o_ref):
    o_ref[...] = g_ref[...] * m_ref[...]


def kernel(x, graphs, kernel):
    nb1 = _N // _ROWS1
    s3 = pl.pallas_call(
        _scores_body,
        grid=(nb1,),
        in_specs=[pl.BlockSpec((_ROWS1, _N), lambda i: (i, 0)),
                  pl.BlockSpec((_N, 1), lambda i: (0, 0))],
        out_specs=pl.BlockSpec((1, _ROWS1, 1), lambda i: (i, 0, 0)),
        out_shape=jax.ShapeDtypeStruct((nb1, _ROWS1, 1), jnp.float32),
        compiler_params=pltpu.CompilerParams(
            dimension_semantics=("parallel",)),
    )(graphs, kernel)
    s = s3.reshape(1, _N)

    mask, xl = pl.pallas_call(
        _mask_xl_body,
        in_specs=[pl.BlockSpec((1, _N), lambda: (0, 0)),
                  pl.BlockSpec((_B, _N), lambda: (0, 0))],
        out_specs=[pl.BlockSpec((1, _N), lambda: (0, 0)),
                   pl.BlockSpec((_B, 1), lambda: (0, 0))],
        out_shape=[jax.ShapeDtypeStruct((1, _N), jnp.float32),
                   jax.ShapeDtypeStruct((_B, 1), jnp.float32)],
    )(s, x)

    nb2 = -(-_N // _ROWS2)
    sub = pl.pallas_call(
        _mask_mul_body,
        grid=(nb2,),
        in_specs=[pl.BlockSpec((_ROWS2, _N), lambda i: (i, 0)),
                  pl.BlockSpec((1, _N), lambda i: (0, 0))],
        out_specs=pl.BlockSpec((_ROWS2, _N), lambda i: (i, 0)),
        out_shape=jax.ShapeDtypeStruct((_N, _N), jnp.float32),
        compiler_params=pltpu.CompilerParams(
            dimension_semantics=("parallel",)),
    )(graphs, mask)
    return (xl, sub)


# speedup vs baseline: 1.0132x; 1.0132x over previous
import jax
import jax.numpy as jnp
from jax.experimental import pallas as pl
from jax.experimental.pallas import tpu as pltpu

_N = 10000
_B = 4
_TOPK = 5000
_ROWS1 = 400
_ROWS2 = 200


def _scores_body(g_ref, k_ref, s_ref):
    s_col = jax.lax.dot_general(
        g_ref[...], k_ref[...], (((1,), (0,)), ((), ())),
        preferred_element_type=jnp.float32)
    s_ref[0, :, :] = jnp.swapaxes(s_col, 0, 1)


def _mask_xl_body(s_ref, x_ref, mask_ref, xl_ref):
    s = s_ref[...]
    x = x_ref[...]
    key = jax.lax.bitcast_convert_type(s, jnp.int32)
    key = jnp.where(key < 0, key ^ jnp.int32(0x7FFFFFFF), key)
    idx = jax.lax.broadcasted_iota(jnp.int32, (1, _N), 1)
    kk = jnp.int32(_TOPK)

    cnt_nonneg = jnp.sum((key >= 0).astype(jnp.int32))
    pos = cnt_nonneg >= kk
    lo0 = jnp.where(pos, jnp.int32(0), jnp.int32(-2147483648))
    hi0 = jnp.where(pos, jnp.int32(2147483647), jnp.int32(-1))

    def body(_, lh):
        lo, hi = lh
        d = hi - lo
        mid = lo + (d >> 1) + (d & 1)
        c = jnp.sum((key >= mid).astype(jnp.int32))
        big = c >= kk
        return (jnp.where(big, mid, lo), jnp.where(big, hi, mid - 1))

    lo, _hi = jax.lax.fori_loop(0, 31, body, (lo0, hi0))
    theta = lo
    gt = key > theta
    eq = key == theta
    need = kk - jnp.sum(gt.astype(jnp.int32))

    def body2(_, lh):
        lo2, hi2 = lh
        mid = lo2 + ((hi2 - lo2) >> 1)
        c = jnp.sum((eq & (idx <= mid)).astype(jnp.int32))
        ok = c >= need
        return (jnp.where(ok, lo2, mid + 1), jnp.where(ok, mid, hi2))

    m, _m2 = jax.lax.fori_loop(0, 14, body2,
                               (jnp.int32(0), jnp.int32(_N - 1)))
    mask = gt | (eq & (idx <= m))
    mask_ref[...] = mask.astype(jnp.float32)
    xl_ref[...] = jnp.sum(x * s, axis=1, keepdims=True)


def _mask_mul_body(g_ref, m_ref, o_ref):
    o_ref[...] = g_ref[...] * m_ref[...]


def kernel(x, graphs, kernel):
    nb1 = _N // _ROWS1
    s3 = pl.pallas_call(
        _scores_body,
        grid=(nb1,),
        in_specs=[pl.BlockSpec((_ROWS1, _N), lambda i: (i, 0)),
                  pl.BlockSpec((_N, 1), lambda i: (0, 0))],
        out_specs=pl.BlockSpec((1, 1, _ROWS1), lambda i: (i, 0, 0)),
        out_shape=jax.ShapeDtypeStruct((nb1, 1, _ROWS1), jnp.float32),
        compiler_params=pltpu.CompilerParams(
            dimension_semantics=("parallel",)),
    )(graphs, kernel)
    s = s3.reshape(1, _N)

    mask, xl = pl.pallas_call(
        _mask_xl_body,
        in_specs=[pl.BlockSpec((1, _N), lambda: (0, 0)),
                  pl.BlockSpec((_B, _N), lambda: (0, 0))],
        out_specs=[pl.BlockSpec((1, _N), lambda: (0, 0)),
                   pl.BlockSpec((_B, 1), lambda: (0, 0))],
        out_shape=[jax.ShapeDtypeStruct((1, _N), jnp.float32),
                   jax.ShapeDtypeStruct((_B, 1), jnp.float32)],
    )(s, x)

    nb2 = -(-_N // _ROWS2)
    sub = pl.pallas_call(
        _mask_mul_body,
        grid=(nb2,),
        in_specs=[pl.BlockSpec((_ROWS2, _N), lambda i: (i, 0)),
                  pl.BlockSpec((1, _N), lambda i: (0, 0))],
        out_specs=pl.BlockSpec((_ROWS2, _N), lambda i: (i, 0)),
        out_shape=jax.ShapeDtypeStruct((_N, _N), jnp.float32),
        compiler_params=pltpu.CompilerParams(
            dimension_semantics=("parallel",)),
    )(graphs, mask)
    return (xl, sub)


# speedup vs baseline: 1.0302x; 1.0168x over previous
import jax
import jax.numpy as jnp
from jax.experimental import pallas as pl
from jax.experimental.pallas import tpu as pltpu

_N = 10000
_B = 4
_TOPK = 5000
_ROWS1 = 400
_ROWS2 = 200


def _scores_body(g_ref, k_ref, s_ref):
    g = g_ref[...].astype(jnp.bfloat16).astype(jnp.float32)
    k = k_ref[...].astype(jnp.bfloat16).astype(jnp.float32)
    s_ref[0, 0, :] = jnp.sum(g * k, axis=1)


def _mask_xl_body(s_ref, x_ref, mask_ref, xl_ref):
    s = s_ref[...]
    x = x_ref[...]
    key = jax.lax.bitcast_convert_type(s, jnp.int32)
    key = jnp.where(key < 0, key ^ jnp.int32(0x7FFFFFFF), key)
    idx = jax.lax.broadcasted_iota(jnp.int32, (1, _N), 1)
    kk = jnp.int32(_TOPK)

    cnt_nonneg = jnp.sum((key >= 0).astype(jnp.int32))
    pos = cnt_nonneg >= kk
    lo0 = jnp.where(pos, jnp.int32(0), jnp.int32(-2147483648))
    hi0 = jnp.where(pos, jnp.int32(2147483647), jnp.int32(-1))

    def body(_, lh):
        lo, hi = lh
        d = hi - lo
        mid = lo + (d >> 1) + (d & 1)
        c = jnp.sum((key >= mid).astype(jnp.int32))
        big = c >= kk
        return (jnp.where(big, mid, lo), jnp.where(big, hi, mid - 1))

    lo, _hi = jax.lax.fori_loop(0, 31, body, (lo0, hi0))
    theta = lo
    gt = key > theta
    eq = key == theta
    need = kk - jnp.sum(gt.astype(jnp.int32))

    def body2(_, lh):
        lo2, hi2 = lh
        mid = lo2 + ((hi2 - lo2) >> 1)
        c = jnp.sum((eq & (idx <= mid)).astype(jnp.int32))
        ok = c >= need
        return (jnp.where(ok, lo2, mid + 1), jnp.where(ok, mid, hi2))

    m, _m2 = jax.lax.fori_loop(0, 14, body2,
                               (jnp.int32(0), jnp.int32(_N - 1)))
    mask = gt | (eq & (idx <= m))
    mask_ref[...] = mask.astype(jnp.float32)
    xl_ref[...] = jnp.sum(x * s, axis=1, keepdims=True)


def _mask_mul_body(g_ref, m_ref, o_ref):
    o_ref[...] = g_ref[...] * m_ref[...]


def kernel(x, graphs, kernel):
    k_row = kernel.reshape(1, _N)
    nb1 = _N // _ROWS1
    s3 = pl.pallas_call(
        _scores_body,
        grid=(nb1,),
        in_specs=[pl.BlockSpec((_ROWS1, _N), lambda i: (i, 0)),
                  pl.BlockSpec((1, _N), lambda i: (0, 0))],
        out_specs=pl.BlockSpec((1, 1, _ROWS1), lambda i: (i, 0, 0)),
        out_shape=jax.ShapeDtypeStruct((nb1, 1, _ROWS1), jnp.float32),
        compiler_params=pltpu.CompilerParams(
            dimension_semantics=("parallel",)),
    )(graphs, k_row)
    s = s3.reshape(1, _N)

    mask, xl = pl.pallas_call(
        _mask_xl_body,
        in_specs=[pl.BlockSpec((1, _N), lambda: (0, 0)),
                  pl.BlockSpec((_B, _N), lambda: (0, 0))],
        out_specs=[pl.BlockSpec((1, _N), lambda: (0, 0)),
                   pl.BlockSpec((_B, 1), lambda: (0, 0))],
        out_shape=[jax.ShapeDtypeStruct((1, _N), jnp.float32),
                   jax.ShapeDtypeStruct((_B, 1), jnp.float32)],
    )(s, x)

    nb2 = -(-_N // _ROWS2)
    sub = pl.pallas_call(
        _mask_mul_body,
        grid=(nb2,),
        in_specs=[pl.BlockSpec((_ROWS2, _N), lambda i: (i, 0)),
                  pl.BlockSpec((1, _N), lambda i: (0, 0))],
        out_specs=pl.BlockSpec((_ROWS2, _N), lambda i: (i, 0)),
        out_shape=jax.ShapeDtypeStruct((_N, _N), jnp.float32),
        compiler_params=pltpu.CompilerParams(
            dimension_semantics=("parallel",)),
    )(graphs, mask)
    return (xl, sub)
